# baseline (speedup 1.0000x reference)
import functools

import jax
import jax.numpy as jnp
from jax import lax
from jax.experimental import pallas as pl
from jax.experimental.pallas import tpu as pltpu
from jax.experimental.pallas import tpu_sc as plsc

_NC = 2
_NS = 16
_NW = _NC * _NS
_B = 128


def _seg_body(chunks, rows_per_tile,
              x_hbm, src_hbm, dst_hbm, zeros_hbm, out_hbm,
              src_v, dst_v, rows_v, acc, sem):
    cid = lax.axis_index("c")
    sid = lax.axis_index("s")
    wid = cid * _NS + sid

    pltpu.sync_copy(zeros_hbm.at[pl.ds(sid * rows_per_tile, rows_per_tile)],
                    acc.at[pl.ds(sid * rows_per_tile, rows_per_tile)])
    pltpu.sync_copy(src_hbm.at[wid], src_v)
    pltpu.sync_copy(dst_hbm.at[wid], dst_v)
    plsc.subcore_barrier()

    def body(c, carry):
        pltpu.async_copy(x_hbm.at[src_v.at[c]], rows_v, sem).wait()
        pltpu.sync_copy(rows_v, acc.at[dst_v.at[c]], add=True)
        return carry

    lax.fori_loop(0, chunks, body, 0)
    plsc.subcore_barrier()

    pltpu.sync_copy(acc.at[pl.ds(sid * rows_per_tile, rows_per_tile)],
                    out_hbm.at[cid, pl.ds(sid * rows_per_tile, rows_per_tile)])


def _sc_segment_sum(x, src3, dst3, zeros):
    _, d = x.shape
    nrows_pad = zeros.shape[0]
    chunks = src3.shape[1]
    rows_per_tile = nrows_pad // _NS
    mesh = plsc.VectorSubcoreMesh(core_axis_name="c", subcore_axis_name="s")
    f = pl.kernel(
        functools.partial(_seg_body, chunks, rows_per_tile),
        out_type=jax.ShapeDtypeStruct((_NC, nrows_pad, d), jnp.float32),
        mesh=mesh,
        scratch_types=[
            pltpu.VMEM((chunks, _B), jnp.int32),
            pltpu.VMEM((chunks, _B), jnp.int32),
            pltpu.VMEM((_B, d), jnp.float32),
            pltpu.VMEM_SHARED((nrows_pad, d), jnp.float32),
            pltpu.SemaphoreType.DMA,
        ],
    )
    return f(x, src3, dst3, zeros)


def _z_body(parts_ref, x_ref, wrel_ref, b2_ref, wroot_ref, z_ref):
    agg = parts_ref[0] + parts_ref[1]
    z_ref[...] = (
        jnp.dot(agg, wrel_ref[...], preferred_element_type=jnp.float32)
        + jnp.dot(x_ref[...], wroot_ref[...], preferred_element_type=jnp.float32)
        + b2_ref[...]
    )


def _zcompute(parts, x, W_rel, b2, W_root):
    n, d = x.shape
    dh = W_rel.shape[1]
    return pl.pallas_call(
        _z_body,
        grid=(1,),
        in_specs=[
            pl.BlockSpec((2, n, d), lambda i: (0, 0, 0)),
            pl.BlockSpec((n, d), lambda i: (0, 0)),
            pl.BlockSpec(W_rel.shape, lambda i: (0, 0)),
            pl.BlockSpec(b2.shape, lambda i: (0, 0)),
            pl.BlockSpec(W_root.shape, lambda i: (0, 0)),
        ],
        out_specs=pl.BlockSpec((n, dh), lambda i: (0, 0)),
        out_shape=jax.ShapeDtypeStruct((n, dh), jnp.float32),
    )(parts, x, W_rel, b2, W_root)


def _dec_body(zr_ref, zf_ref, o_ref):
    logits = lax.dot_general(
        zr_ref[...], zf_ref[...], (((1,), (1,)), ((), ())),
        preferred_element_type=jnp.float32,
    )
    o_ref[...] = jax.nn.sigmoid(logits)


def _decoder(z, rows_blk):
    n, dh = z.shape
    grid = (n // rows_blk,)
    return pl.pallas_call(
        _dec_body,
        grid=grid,
        in_specs=[
            pl.BlockSpec((rows_blk, dh), lambda i: (i, 0)),
            pl.BlockSpec((n, dh), lambda i: (0, 0)),
        ],
        out_specs=pl.BlockSpec((rows_blk, n), lambda i: (i, 0)),
        out_shape=jax.ShapeDtypeStruct((n, n), jnp.float32),
    )(z, z)


def kernel(x, edge_index, W_rel, b_rel, W_root):
    n, d = x.shape
    dh = W_rel.shape[1]
    e = edge_index.shape[1]

    chunks = -(-e // (_NW * _B))
    e_pad = _NW * chunks * _B
    src = edge_index[0]
    dst = edge_index[1]
    if e_pad != e:
        pad = e_pad - e
        src = jnp.concatenate([src, jnp.zeros((pad,), jnp.int32)])
        dst = jnp.concatenate([dst, jnp.full((pad,), n, jnp.int32)])
    src3 = src.reshape(_NW, chunks, _B)
    dst3 = dst.reshape(_NW, chunks, _B)
    nrows_pad = -(-(n + 1) // (_NS * 8)) * (_NS * 8)
    zeros = jnp.zeros((nrows_pad, d), jnp.float32)

    parts = _sc_segment_sum(x, src3, dst3, zeros)
    z = _zcompute(parts, x, W_rel, b_rel.reshape(1, dh), W_root)
    adj = _decoder(z, 200)
    return adj, z


# speedup vs baseline: 1.0170x; 1.0170x over previous
import functools

import jax
import jax.numpy as jnp
from jax import lax
from jax.experimental import pallas as pl
from jax.experimental.pallas import tpu as pltpu
from jax.experimental.pallas import tpu_sc as plsc

_NC = 2
_NS = 16
_NW = _NC * _NS
_B = 128


def _seg_body(chunks, rows_per_tile,
              x_hbm, src_hbm, dst_hbm, zeros_hbm, out_hbm,
              src_v, dst_v, rows_v, acc, sem):
    cid = lax.axis_index("c")
    sid = lax.axis_index("s")
    wid = cid * _NS + sid

    pltpu.sync_copy(zeros_hbm.at[pl.ds(sid * rows_per_tile, rows_per_tile)],
                    acc.at[pl.ds(sid * rows_per_tile, rows_per_tile)])
    pltpu.sync_copy(src_hbm.at[wid], src_v)
    pltpu.sync_copy(dst_hbm.at[wid], dst_v)
    plsc.subcore_barrier()

    def body(c, carry):
        pltpu.async_copy(x_hbm.at[src_v.at[c]], rows_v, sem).wait()
        pltpu.sync_copy(rows_v, acc.at[dst_v.at[c]], add=True)
        return carry

    lax.fori_loop(0, chunks, body, 0)
    plsc.subcore_barrier()

    pltpu.sync_copy(acc.at[pl.ds(sid * rows_per_tile, rows_per_tile)],
                    out_hbm.at[cid, pl.ds(sid * rows_per_tile, rows_per_tile)])


def _sc_segment_sum(x, src3, dst3, zeros):
    _, d = x.shape
    nrows_pad = zeros.shape[0]
    chunks = src3.shape[1]
    rows_per_tile = nrows_pad // _NS
    mesh = plsc.VectorSubcoreMesh(core_axis_name="c", subcore_axis_name="s")
    f = pl.kernel(
        functools.partial(_seg_body, chunks, rows_per_tile),
        out_type=jax.ShapeDtypeStruct((_NC, nrows_pad, d), jnp.float32),
        mesh=mesh,
        scratch_types=[
            pltpu.VMEM((chunks, _B), jnp.int32),
            pltpu.VMEM((chunks, _B), jnp.int32),
            pltpu.VMEM((_B, d), jnp.float32),
            pltpu.VMEM_SHARED((nrows_pad, d), jnp.float32),
            pltpu.SemaphoreType.DMA,
        ],
    )
    return f(x, src3, dst3, zeros)


def _z_body(parts_ref, x_ref, wrel_ref, b2_ref, wroot_ref, z_ref):
    agg = parts_ref[0] + parts_ref[1]
    z_ref[...] = (
        jnp.dot(agg, wrel_ref[...], preferred_element_type=jnp.float32)
        + jnp.dot(x_ref[...], wroot_ref[...], preferred_element_type=jnp.float32)
        + b2_ref[...]
    )


def _zcompute(parts, x, W_rel, b2, W_root):
    n, d = x.shape
    dh = W_rel.shape[1]
    return pl.pallas_call(
        _z_body,
        grid=(1,),
        in_specs=[
            pl.BlockSpec((2, n, d), lambda i: (0, 0, 0)),
            pl.BlockSpec((n, d), lambda i: (0, 0)),
            pl.BlockSpec(W_rel.shape, lambda i: (0, 0)),
            pl.BlockSpec(b2.shape, lambda i: (0, 0)),
            pl.BlockSpec(W_root.shape, lambda i: (0, 0)),
        ],
        out_specs=pl.BlockSpec((n, dh), lambda i: (0, 0)),
        out_shape=jax.ShapeDtypeStruct((n, dh), jnp.float32),
    )(parts, x, W_rel, b2, W_root)


def _dec_body(zr_ref, zf_ref, o_ref):
    logits = lax.dot_general(
        zr_ref[...], zf_ref[...], (((1,), (1,)), ((), ())),
        preferred_element_type=jnp.float32,
    )
    o_ref[...] = jax.nn.sigmoid(logits)


def _decoder(z, rows_blk):
    n, dh = z.shape
    grid = (n // rows_blk,)
    return pl.pallas_call(
        _dec_body,
        grid=grid,
        in_specs=[
            pl.BlockSpec((rows_blk, dh), lambda i: (i, 0)),
            pl.BlockSpec((n, dh), lambda i: (0, 0)),
        ],
        out_specs=pl.BlockSpec((rows_blk, n), lambda i: (i, 0)),
        out_shape=jax.ShapeDtypeStruct((n, n), jnp.float32),
    )(z, z)


def kernel(x, edge_index, W_rel, b_rel, W_root):
    n, d = x.shape
    dh = W_rel.shape[1]
    e = edge_index.shape[1]

    chunks = -(-e // (_NW * _B))
    e_pad = _NW * chunks * _B
    src = edge_index[0]
    dst = edge_index[1]
    if e_pad != e:
        pad = e_pad - e
        src = jnp.concatenate([src, jnp.zeros((pad,), jnp.int32)])
        dst = jnp.concatenate([dst, jnp.full((pad,), n, jnp.int32)])
    src3 = src.reshape(_NW, chunks, _B)
    dst3 = dst.reshape(_NW, chunks, _B)
    nrows_pad = -(-(n + 1) // (_NS * 8)) * (_NS * 8)
    zeros = jnp.zeros((nrows_pad, d), jnp.float32)

    parts = _sc_segment_sum(x, src3, dst3, zeros)
    z = _zcompute(parts, x, W_rel, b_rel.reshape(1, dh), W_root)
    adj = _decoder(z, 400)
    return adj, z
